# Initial kernel scaffold; baseline (speedup 1.0000x reference)
import jax
import jax.numpy as jnp
from jax.experimental import pallas as pl


def kernel(edge_index, w1, b1, w2, b2, w3, b3, w4, b4, w5, b5, l1_w, l1_b, l2_w, l2_b):
    raise NotImplementedError("write your pallas kernel here")


# speedup vs baseline: 5.3270x; 5.3270x over previous
import functools

import jax
import jax.numpy as jnp
from jax import lax
from jax.experimental import pallas as pl
from jax.experimental.pallas import tpu as pltpu
from jax.experimental.pallas import tpu_sc as plsc

N_NODES = 10000
N_EDGES = 320000
HID = 128
HID2 = 64
W16 = 16

NC = 2
NS = 16
NW = NC * NS
CHUNK = 128
CH_PER_W = 79
EDGES_PER_W = CHUNK * CH_PER_W
E_PAD = NW * EDGES_PER_W
NROW = NS * 640
ROWS_PER_TILE = NROW // NS

_mesh = plsc.VectorSubcoreMesh(core_axis_name="c", subcore_axis_name="s")


def _zero_fill(ref, rows, width):
    z = jnp.zeros((16,), jnp.float32)
    for i in range(rows):
        for k in range(width // 16):
            ref[i, pl.ds(16 * k, 16)] = z


DCH = 64


@functools.partial(
    pl.kernel,
    out_type=jax.ShapeDtypeStruct((NC, NROW, HID), jnp.float32),
    mesh=_mesh,
    scratch_types=[
        pltpu.VMEM((DCH,), jnp.int32),
        pltpu.VMEM((DCH,), jnp.int32),
        pltpu.VMEM((DCH, HID), jnp.float32),
        pltpu.VMEM((DCH, HID), jnp.float32),
        pltpu.VMEM((DCH, HID), jnp.float32),
        pltpu.VMEM_SHARED((NROW, HID), jnp.float32),
    ],
)
def _deg_kernel(src_hbm, dst_hbm, deg_hbm,
                sbuf, dbuf, ones_lo, ones_hi, stg, acc):
    cid = lax.axis_index("c")
    sid = lax.axis_index("s")
    wid = cid * NS + sid
    onev = jnp.ones((16,), jnp.float32)
    zerov = jnp.zeros((16,), jnp.float32)
    for i in range(DCH):
        for k in range(HID // 16):
            ones_lo[i, pl.ds(16 * k, 16)] = onev if k < 4 else zerov
            ones_hi[i, pl.ds(16 * k, 16)] = zerov if k < 4 else onev
    _zero_fill(stg, DCH, HID)
    base = sid * ROWS_PER_TILE
    for t in range(ROWS_PER_TILE // DCH):
        pltpu.sync_copy(stg, acc.at[pl.ds(base + t * DCH, DCH)])
    plsc.subcore_barrier()

    def step(j, carry):
        pltpu.sync_copy(dst_hbm.at[wid, j], dbuf)
        pltpu.sync_copy(ones_lo, acc.at[dbuf], add=True)
        pltpu.sync_copy(src_hbm.at[wid, j], sbuf)
        pltpu.sync_copy(ones_hi, acc.at[sbuf], add=True)
        return carry

    lax.fori_loop(0, CH_PER_W * (CHUNK // DCH), step, 0)
    plsc.subcore_barrier()
    for t in range(ROWS_PER_TILE // DCH):
        pltpu.sync_copy(acc.at[pl.ds(base + t * DCH, DCH)], stg)
        pltpu.sync_copy(stg, deg_hbm.at[cid, pl.ds(base + t * DCH, DCH)])


def _make_agg_kernel(D):
    @functools.partial(
        pl.kernel,
        out_type=jax.ShapeDtypeStruct((NC, NROW, D), jnp.float32),
        mesh=_mesh,
        scratch_types=[
            pltpu.VMEM((CHUNK,), jnp.int32),
            pltpu.VMEM((CHUNK,), jnp.int32),
            pltpu.VMEM((CHUNK, D), jnp.float32),
            pltpu.VMEM_SHARED((NROW, D), jnp.float32),
        ],
    )
    def _agg(src_hbm, dst_hbm, xs_hbm, out_hbm, sbuf, dbuf, gbuf, acc):
        cid = lax.axis_index("c")
        sid = lax.axis_index("s")
        wid = cid * NS + sid
        _zero_fill(gbuf, CHUNK, D)
        base = sid * ROWS_PER_TILE
        for t in range(ROWS_PER_TILE // CHUNK):
            pltpu.sync_copy(gbuf, acc.at[pl.ds(base + t * CHUNK, CHUNK)])
        plsc.subcore_barrier()

        def step(j, carry):
            pltpu.sync_copy(src_hbm.at[wid, j], sbuf)
            pltpu.sync_copy(dst_hbm.at[wid, j], dbuf)
            pltpu.sync_copy(xs_hbm.at[sbuf], gbuf)
            pltpu.sync_copy(gbuf, acc.at[dbuf], add=True)
            return carry

        lax.fori_loop(0, CH_PER_W, step, 0)
        plsc.subcore_barrier()
        for t in range(ROWS_PER_TILE // CHUNK):
            pltpu.sync_copy(acc.at[pl.ds(base + t * CHUNK, CHUNK)], gbuf)
            pltpu.sync_copy(gbuf, out_hbm.at[cid, pl.ds(base + t * CHUNK, CHUNK)])

    return _agg


_agg128 = _make_agg_kernel(HID)


def _prep_body(degp_ref, w1p_ref, xs1_ref, nd_ref, ns_ref):
    d = degp_ref[0] + degp_ref[1]
    di = jnp.broadcast_to(d[:, 0:1], (NROW, W16))
    do = jnp.broadcast_to(d[:, 64:65], (NROW, W16))
    rows = lax.broadcasted_iota(jnp.int32, di.shape, 0)
    valid = rows < N_NODES
    nsv = jnp.where(valid & (do > 0.0), lax.rsqrt(do), 0.0)
    ndv = jnp.where(valid & (di > 0.0), lax.rsqrt(di), 0.0)
    lane = lax.broadcasted_iota(jnp.int32, di.shape, 1)
    one = jnp.float32(1.0)
    zero = jnp.float32(0.0)
    h = jnp.where(
        lane == 0, di,
        jnp.where(lane == 1, jnp.where(di > 3.0, one, zero),
                  jnp.where(lane == 2, 3.0 / di,
                            jnp.where(lane == 3, jnp.where(di > 4.0, one, zero),
                                      zero))))
    hs = jnp.where(valid, h * nsv, 0.0)
    xs1_ref[...] = jnp.dot(hs, w1p_ref[...], preferred_element_type=jnp.float32)
    nd_ref[...] = ndv
    ns_ref[...] = nsv


_prep_tc = pl.pallas_call(
    _prep_body,
    out_shape=(
        jax.ShapeDtypeStruct((NROW, HID), jnp.float32),
        jax.ShapeDtypeStruct((NROW, W16), jnp.float32),
        jax.ShapeDtypeStruct((NROW, W16), jnp.float32),
    ),
)


def _layer1_body(p_ref, nd_ref, ns_ref, b_ref, xs_ref):
    agg = (p_ref[0] + p_ref[1]) * nd_ref[:, 0:1]
    xs_ref[...] = jnp.tanh(agg + b_ref[...]) * ns_ref[:, 0:1]


_layer1_tc = pl.pallas_call(
    _layer1_body,
    out_shape=jax.ShapeDtypeStruct((NROW, HID), jnp.float32),
)


def _layer_body(p_ref, nd_ref, ns_ref, w_ref, b_ref, xs_ref):
    agg = (p_ref[0] + p_ref[1]) * nd_ref[:, 0:1]
    h = jnp.tanh(jnp.dot(agg, w_ref[...],
                         preferred_element_type=jnp.float32) + b_ref[...])
    xs_ref[...] = h * ns_ref[:, 0:1]


_layer_tc128 = pl.pallas_call(
    _layer_body,
    out_shape=jax.ShapeDtypeStruct((NROW, HID), jnp.float32),
)


def _final_body(p_ref, nd_ref, w_ref, b_ref, l1w_ref, l1b_ref, l2w_ref,
                l2b_ref, h_ref, g_ref, pred_ref):
    agg = (p_ref[0] + p_ref[1]) * nd_ref[:, 0:1]
    h = jnp.tanh(jnp.dot(agg, w_ref[...],
                         preferred_element_type=jnp.float32) + b_ref[...])
    h_ref[...] = h
    rows = lax.broadcasted_iota(jnp.int32, h.shape, 0)
    hm = jnp.where(rows < N_NODES, h, 0.0)
    g = jnp.sum(hm, axis=0, keepdims=True) * jnp.float32(1.0 / N_NODES)
    g_ref[...] = g
    e = jnp.dot(g, l1w_ref[...], preferred_element_type=jnp.float32) + l1b_ref[...]
    e = jnp.where(e > 0.0, e, 0.01 * e)
    z = jnp.sum(e * l2w_ref[...]) + l2b_ref[0, 0]
    pred_ref[...] = jnp.reshape(1.0 / (1.0 + jnp.exp(-z)), (1, 1))


_final_tc = pl.pallas_call(
    _final_body,
    out_shape=(
        jax.ShapeDtypeStruct((NROW, HID), jnp.float32),
        jax.ShapeDtypeStruct((1, HID), jnp.float32),
        jax.ShapeDtypeStruct((1, 1), jnp.float32),
    ),
)


def kernel(edge_index, w1, b1, w2, b2, w3, b3, w4, b4, w5, b5,
           l1_w, l1_b, l2_w, l2_b):
    src = edge_index[0]
    dst = edge_index[1]
    n_pad = E_PAD - N_EDGES
    pad_idx = (jnp.arange(n_pad, dtype=jnp.int32) % (NROW - N_NODES)) + N_NODES
    src_rs = jnp.concatenate([src, pad_idx]).reshape(NW, CH_PER_W, CHUNK)
    dst_rs = jnp.concatenate([dst, pad_idx]).reshape(NW, CH_PER_W, CHUNK)
    src_rs64 = src_rs.reshape(NW, CH_PER_W * (CHUNK // DCH), DCH)
    dst_rs64 = dst_rs.reshape(NW, CH_PER_W * (CHUNK // DCH), DCH)

    w1p = jnp.zeros((W16, HID), jnp.float32).at[:4].set(w1)
    b1r = b1.reshape(1, HID)
    b2r = b2.reshape(1, HID)
    b3r = b3.reshape(1, HID)
    b4r = b4.reshape(1, HID)
    b5r = b5.reshape(1, HID)
    l1br = l1_b.reshape(1, HID2)
    l2wr = l2_w.reshape(1, HID2)
    l2br = l2_b.reshape(1, 1)

    deg_p = _deg_kernel(src_rs64, dst_rs64)
    xs1, nd, ns = _prep_tc(deg_p, w1p)

    agg1 = _agg128(src_rs, dst_rs, xs1)
    xs2 = _layer1_tc(agg1, nd, ns, b1r)
    agg2 = _agg128(src_rs, dst_rs, xs2)
    xs3 = _layer_tc128(agg2, nd, ns, w2, b2r)
    agg3 = _agg128(src_rs, dst_rs, xs3)
    xs4 = _layer_tc128(agg3, nd, ns, w3, b3r)
    agg4 = _agg128(src_rs, dst_rs, xs4)
    xs5 = _layer_tc128(agg4, nd, ns, w4, b4r)
    agg5 = _agg128(src_rs, dst_rs, xs5)
    h_full, graph_emb, pred = _final_tc(agg5, nd, w5, b5r, l1_w, l1br,
                                        l2wr, l2br)
    h_co = h_full[:N_NODES]
    return (pred, graph_emb, h_co)


# speedup vs baseline: 6.9675x; 1.3080x over previous
import functools

import jax
import jax.numpy as jnp
from jax import lax
from jax.experimental import pallas as pl
from jax.experimental.pallas import tpu as pltpu
from jax.experimental.pallas import tpu_sc as plsc

N_NODES = 10000
N_EDGES = 320000
HID = 128
HID2 = 64
W16 = 16

NC = 2
NS = 16
NW = NC * NS
CHUNK = 128
CH_PER_W = 79
EDGES_PER_W = CHUNK * CH_PER_W
E_PAD = NW * EDGES_PER_W
NROW = NS * 640
ROWS_PER_TILE = NROW // NS

_mesh = plsc.VectorSubcoreMesh(core_axis_name="c", subcore_axis_name="s")


def _zero_fill(ref, rows, width):
    z = jnp.zeros((16,), jnp.float32)
    for i in range(rows):
        for k in range(width // 16):
            ref[i, pl.ds(16 * k, 16)] = z


DCH = 64


@functools.partial(
    pl.kernel,
    out_type=jax.ShapeDtypeStruct((NC, NROW, HID), jnp.float32),
    mesh=_mesh,
    scratch_types=[
        pltpu.VMEM((DCH,), jnp.int32),
        pltpu.VMEM((DCH,), jnp.int32),
        pltpu.VMEM((DCH, HID), jnp.float32),
        pltpu.VMEM((DCH, HID), jnp.float32),
        pltpu.VMEM((DCH, HID), jnp.float32),
        pltpu.VMEM_SHARED((NROW, HID), jnp.float32),
    ],
)
def _deg_kernel(src_hbm, dst_hbm, deg_hbm,
                sbuf, dbuf, ones_lo, ones_hi, stg, acc):
    cid = lax.axis_index("c")
    sid = lax.axis_index("s")
    wid = cid * NS + sid
    onev = jnp.ones((16,), jnp.float32)
    zerov = jnp.zeros((16,), jnp.float32)
    for i in range(DCH):
        for k in range(HID // 16):
            ones_lo[i, pl.ds(16 * k, 16)] = onev if k < 4 else zerov
            ones_hi[i, pl.ds(16 * k, 16)] = zerov if k < 4 else onev
    _zero_fill(stg, DCH, HID)
    base = sid * ROWS_PER_TILE
    for t in range(ROWS_PER_TILE // DCH):
        pltpu.sync_copy(stg, acc.at[pl.ds(base + t * DCH, DCH)])
    plsc.subcore_barrier()

    nch = CH_PER_W * (CHUNK // DCH)

    def step(j, carry):
        off = (wid * nch + j) * DCH
        pltpu.sync_copy(dst_hbm.at[pl.ds(off, DCH)], dbuf)
        pltpu.sync_copy(ones_lo, acc.at[dbuf], add=True)
        pltpu.sync_copy(src_hbm.at[pl.ds(off, DCH)], sbuf)
        pltpu.sync_copy(ones_hi, acc.at[sbuf], add=True)
        return carry

    lax.fori_loop(0, nch, step, 0)
    plsc.subcore_barrier()
    for t in range(ROWS_PER_TILE // DCH):
        pltpu.sync_copy(acc.at[pl.ds(base + t * DCH, DCH)], stg)
        pltpu.sync_copy(stg, deg_hbm.at[cid, pl.ds(base + t * DCH, DCH)])


def _make_agg_kernel(D):
    @functools.partial(
        pl.kernel,
        out_type=jax.ShapeDtypeStruct((NC, NROW, D), jnp.float32),
        mesh=_mesh,
        scratch_types=[
            pltpu.VMEM((CHUNK,), jnp.int32),
            pltpu.VMEM((CHUNK,), jnp.int32),
            pltpu.VMEM((CHUNK,), jnp.int32),
            pltpu.VMEM((CHUNK,), jnp.int32),
            pltpu.VMEM((CHUNK, D), jnp.float32),
            pltpu.VMEM((CHUNK, D), jnp.float32),
            pltpu.SemaphoreType.DMA,
            pltpu.SemaphoreType.DMA,
            pltpu.SemaphoreType.DMA,
            pltpu.SemaphoreType.DMA,
            pltpu.SemaphoreType.DMA,
            pltpu.SemaphoreType.DMA,
            pltpu.VMEM_SHARED((NROW, D), jnp.float32),
        ],
    )
    def _agg(src_hbm, dst_hbm, xs_hbm, out_hbm,
             sb0, sb1, db0, db1, gb0, gb1, ss0, ss1, ds0, ds1, gs0, gs1, acc):
        cid = lax.axis_index("c")
        sid = lax.axis_index("s")
        wid = cid * NS + sid
        _zero_fill(gb0, CHUNK, D)
        base = sid * ROWS_PER_TILE
        for t in range(ROWS_PER_TILE // CHUNK):
            pltpu.sync_copy(gb0, acc.at[pl.ds(base + t * CHUNK, CHUNK)])
        plsc.subcore_barrier()

        def pair(jj, carry):
            j0 = (wid * CH_PER_W + 2 * jj) * CHUNK
            j1 = j0 + CHUNK
            pltpu.sync_copy(src_hbm.at[pl.ds(j0, CHUNK)], sb0)
            pltpu.sync_copy(dst_hbm.at[pl.ds(j0, CHUNK)], db0)
            g0 = pltpu.async_copy(xs_hbm.at[sb0], gb0, gs0)
            pltpu.sync_copy(src_hbm.at[pl.ds(j1, CHUNK)], sb1)
            pltpu.sync_copy(dst_hbm.at[pl.ds(j1, CHUNK)], db1)
            g1 = pltpu.async_copy(xs_hbm.at[sb1], gb1, gs1)
            g0.wait()
            s0 = pltpu.async_copy(gb0, acc.at[db0], ss0, add=True)
            g1.wait()
            s1 = pltpu.async_copy(gb1, acc.at[db1], ss1, add=True)
            s0.wait()
            s1.wait()
            return carry

        lax.fori_loop(0, CH_PER_W // 2, pair, 0)
        off = (wid * CH_PER_W + CH_PER_W - 1) * CHUNK
        pltpu.sync_copy(src_hbm.at[pl.ds(off, CHUNK)], sb0)
        pltpu.sync_copy(dst_hbm.at[pl.ds(off, CHUNK)], db0)
        pltpu.sync_copy(xs_hbm.at[sb0], gb0)
        pltpu.sync_copy(gb0, acc.at[db0], add=True)
        plsc.subcore_barrier()
        for t in range(ROWS_PER_TILE // CHUNK):
            pltpu.sync_copy(acc.at[pl.ds(base + t * CHUNK, CHUNK)], gb0)
            pltpu.sync_copy(gb0, out_hbm.at[cid, pl.ds(base + t * CHUNK, CHUNK)])

    return _agg


_agg128 = _make_agg_kernel(HID)


def _prep_body(degp_ref, w1p_ref, xs1_ref, nd_ref, ns_ref):
    d = degp_ref[0] + degp_ref[1]
    di = jnp.broadcast_to(d[:, 0:1], (NROW, W16))
    do = jnp.broadcast_to(d[:, 64:65], (NROW, W16))
    rows = lax.broadcasted_iota(jnp.int32, di.shape, 0)
    valid = rows < N_NODES
    nsv = jnp.where(valid & (do > 0.0), lax.rsqrt(do), 0.0)
    ndv = jnp.where(valid & (di > 0.0), lax.rsqrt(di), 0.0)
    lane = lax.broadcasted_iota(jnp.int32, di.shape, 1)
    one = jnp.float32(1.0)
    zero = jnp.float32(0.0)
    h = jnp.where(
        lane == 0, di,
        jnp.where(lane == 1, jnp.where(di > 3.0, one, zero),
                  jnp.where(lane == 2, 3.0 / di,
                            jnp.where(lane == 3, jnp.where(di > 4.0, one, zero),
                                      zero))))
    hs = jnp.where(valid, h * nsv, 0.0)
    xs1_ref[...] = jnp.dot(hs, w1p_ref[...], preferred_element_type=jnp.float32)
    nd_ref[...] = ndv
    ns_ref[...] = nsv


_prep_tc = pl.pallas_call(
    _prep_body,
    out_shape=(
        jax.ShapeDtypeStruct((NROW, HID), jnp.float32),
        jax.ShapeDtypeStruct((NROW, W16), jnp.float32),
        jax.ShapeDtypeStruct((NROW, W16), jnp.float32),
    ),
)


def _layer1_body(p_ref, nd_ref, ns_ref, b_ref, xs_ref):
    agg = (p_ref[0] + p_ref[1]) * nd_ref[:, 0:1]
    xs_ref[...] = jnp.tanh(agg + b_ref[...]) * ns_ref[:, 0:1]


_layer1_tc = pl.pallas_call(
    _layer1_body,
    out_shape=jax.ShapeDtypeStruct((NROW, HID), jnp.float32),
)


def _layer_body(p_ref, nd_ref, ns_ref, w_ref, b_ref, xs_ref):
    agg = (p_ref[0] + p_ref[1]) * nd_ref[:, 0:1]
    h = jnp.tanh(jnp.dot(agg, w_ref[...],
                         preferred_element_type=jnp.float32) + b_ref[...])
    xs_ref[...] = h * ns_ref[:, 0:1]


_layer_tc128 = pl.pallas_call(
    _layer_body,
    out_shape=jax.ShapeDtypeStruct((NROW, HID), jnp.float32),
)


def _final_body(p_ref, nd_ref, w_ref, b_ref, l1w_ref, l1b_ref, l2w_ref,
                l2b_ref, h_ref, g_ref, pred_ref):
    agg = (p_ref[0] + p_ref[1]) * nd_ref[:, 0:1]
    h = jnp.tanh(jnp.dot(agg, w_ref[...],
                         preferred_element_type=jnp.float32) + b_ref[...])
    h_ref[...] = h
    rows = lax.broadcasted_iota(jnp.int32, h.shape, 0)
    hm = jnp.where(rows < N_NODES, h, 0.0)
    g = jnp.sum(hm, axis=0, keepdims=True) * jnp.float32(1.0 / N_NODES)
    g_ref[...] = g
    e = jnp.dot(g, l1w_ref[...], preferred_element_type=jnp.float32) + l1b_ref[...]
    e = jnp.where(e > 0.0, e, 0.01 * e)
    z = jnp.sum(e * l2w_ref[...]) + l2b_ref[0, 0]
    pred_ref[...] = jnp.reshape(1.0 / (1.0 + jnp.exp(-z)), (1, 1))


_final_tc = pl.pallas_call(
    _final_body,
    out_shape=(
        jax.ShapeDtypeStruct((NROW, HID), jnp.float32),
        jax.ShapeDtypeStruct((1, HID), jnp.float32),
        jax.ShapeDtypeStruct((1, 1), jnp.float32),
    ),
)


def kernel(edge_index, w1, b1, w2, b2, w3, b3, w4, b4, w5, b5,
           l1_w, l1_b, l2_w, l2_b):
    src = edge_index[0]
    dst = edge_index[1]
    n_pad = E_PAD - N_EDGES
    pad_idx = (jnp.arange(n_pad, dtype=jnp.int32) % (NROW - N_NODES)) + N_NODES
    src_fl = jnp.concatenate([src, pad_idx])
    dst_fl = jnp.concatenate([dst, pad_idx])

    w1p = jnp.zeros((W16, HID), jnp.float32).at[:4].set(w1)
    b1r = b1.reshape(1, HID)
    b2r = b2.reshape(1, HID)
    b3r = b3.reshape(1, HID)
    b4r = b4.reshape(1, HID)
    b5r = b5.reshape(1, HID)
    l1br = l1_b.reshape(1, HID2)
    l2wr = l2_w.reshape(1, HID2)
    l2br = l2_b.reshape(1, 1)

    deg_p = _deg_kernel(src_fl, dst_fl)
    xs1, nd, ns = _prep_tc(deg_p, w1p)

    agg1 = _agg128(src_fl, dst_fl, xs1)
    xs2 = _layer1_tc(agg1, nd, ns, b1r)
    agg2 = _agg128(src_fl, dst_fl, xs2)
    xs3 = _layer_tc128(agg2, nd, ns, w2, b2r)
    agg3 = _agg128(src_fl, dst_fl, xs3)
    xs4 = _layer_tc128(agg3, nd, ns, w3, b3r)
    agg4 = _agg128(src_fl, dst_fl, xs4)
    xs5 = _layer_tc128(agg4, nd, ns, w4, b4r)
    agg5 = _agg128(src_fl, dst_fl, xs5)
    h_full, graph_emb, pred = _final_tc(agg5, nd, w5, b5r, l1_w, l1br,
                                        l2wr, l2br)
    h_co = h_full[:N_NODES]
    return (pred, graph_emb, h_co)


# speedup vs baseline: 7.9493x; 1.1409x over previous
import functools

import jax
import jax.numpy as jnp
from jax import lax
from jax.experimental import pallas as pl
from jax.experimental.pallas import tpu as pltpu
from jax.experimental.pallas import tpu_sc as plsc

N_NODES = 10000
N_EDGES = 320000
HID = 128
HID2 = 64
W16 = 16

NC = 2
NS = 16
NW = NC * NS
CHUNK = 128
CH_PER_W = 80
EDGES_PER_W = CHUNK * CH_PER_W
E_PAD = NW * EDGES_PER_W
NROW = NS * 640
ROWS_PER_TILE = NROW // NS

_mesh = plsc.VectorSubcoreMesh(core_axis_name="c", subcore_axis_name="s")


def _zero_fill(ref, rows, width):
    z = jnp.zeros((16,), jnp.float32)
    for i in range(rows):
        for k in range(width // 16):
            ref[i, pl.ds(16 * k, 16)] = z


def _unpack_idx(slab, j, buf):
    for k in range(CHUNK // 32):
        w = slab[pl.ds(j * (CHUNK // 2) + 16 * k, 16)]
        buf[pl.ds(32 * k, 16)] = w & 0xFFFF
        buf[pl.ds(32 * k + 16, 16)] = lax.shift_right_logical(w, 16)


@functools.partial(
    pl.kernel,
    out_type=jax.ShapeDtypeStruct((NC, NROW, HID), jnp.float32),
    mesh=_mesh,
    scratch_types=[
        pltpu.VMEM((EDGES_PER_W // 2,), jnp.int32),
        pltpu.VMEM((EDGES_PER_W // 2,), jnp.int32),
        pltpu.VMEM((CHUNK,), jnp.int32),
        pltpu.VMEM((CHUNK,), jnp.int32),
        pltpu.VMEM((CHUNK,), jnp.int32),
        pltpu.VMEM((CHUNK,), jnp.int32),
        pltpu.VMEM((CHUNK, HID), jnp.float32),
        pltpu.VMEM((CHUNK, HID), jnp.float32),
        pltpu.VMEM((16, HID), jnp.float32),
        pltpu.SemaphoreType.DMA,
        pltpu.SemaphoreType.DMA,
        pltpu.SemaphoreType.DMA,
        pltpu.SemaphoreType.DMA,
        pltpu.VMEM_SHARED((NROW, HID), jnp.float32),
    ],
)
def _deg_kernel(src16_hbm, dst16_hbm, deg_hbm,
                s16, d16, sb0, sb1, db0, db1, ones_lo, ones_hi, zb,
                m0, m1, m2, m3, acc):
    cid = lax.axis_index("c")
    sid = lax.axis_index("s")
    wid = cid * NS + sid
    half = EDGES_PER_W // 2
    pltpu.sync_copy(src16_hbm.at[pl.ds(wid * half, half)], s16)
    pltpu.sync_copy(dst16_hbm.at[pl.ds(wid * half, half)], d16)
    onev = jnp.ones((16,), jnp.float32)
    zerov = jnp.zeros((16,), jnp.float32)
    for i in range(CHUNK):
        for k in range(HID // 16):
            ones_lo[i, pl.ds(16 * k, 16)] = onev if k < 4 else zerov
            ones_hi[i, pl.ds(16 * k, 16)] = zerov if k < 4 else onev
    _zero_fill(zb, 16, HID)
    base = sid * ROWS_PER_TILE
    for t in range(ROWS_PER_TILE // 16):
        pltpu.sync_copy(zb, acc.at[pl.ds(base + t * 16, 16)])
    plsc.subcore_barrier()

    def pair(jj, carry):
        j0 = 2 * jj
        _unpack_idx(d16, j0, db0)
        a0 = pltpu.async_copy(ones_lo, acc.at[db0], m0, add=True)
        _unpack_idx(s16, j0, sb0)
        b0 = pltpu.async_copy(ones_hi, acc.at[sb0], m1, add=True)
        _unpack_idx(d16, j0 + 1, db1)
        a1 = pltpu.async_copy(ones_lo, acc.at[db1], m2, add=True)
        _unpack_idx(s16, j0 + 1, sb1)
        b1 = pltpu.async_copy(ones_hi, acc.at[sb1], m3, add=True)
        a0.wait()
        b0.wait()
        a1.wait()
        b1.wait()
        return carry

    lax.fori_loop(0, CH_PER_W // 2, pair, 0)
    plsc.subcore_barrier()
    for t in range(ROWS_PER_TILE // CHUNK):
        pltpu.sync_copy(acc.at[pl.ds(base + t * CHUNK, CHUNK)], ones_lo)
        pltpu.sync_copy(ones_lo, deg_hbm.at[cid, pl.ds(base + t * CHUNK, CHUNK)])


@functools.partial(
    pl.kernel,
    out_type=jax.ShapeDtypeStruct((NC, NROW, HID), jnp.float32),
    mesh=_mesh,
    scratch_types=[
        pltpu.VMEM((EDGES_PER_W // 2,), jnp.int32),
        pltpu.VMEM((EDGES_PER_W // 2,), jnp.int32),
        pltpu.VMEM((CHUNK,), jnp.int32),
        pltpu.VMEM((CHUNK,), jnp.int32),
        pltpu.VMEM((CHUNK,), jnp.int32),
        pltpu.VMEM((CHUNK,), jnp.int32),
        pltpu.VMEM((CHUNK, HID), jnp.float32),
        pltpu.VMEM((CHUNK, HID), jnp.float32),
        pltpu.SemaphoreType.DMA,
        pltpu.SemaphoreType.DMA,
        pltpu.SemaphoreType.DMA,
        pltpu.SemaphoreType.DMA,
        pltpu.VMEM_SHARED((NROW, HID), jnp.float32),
    ],
)
def _agg128(src16_hbm, dst16_hbm, xs_hbm, out_hbm,
            s16, d16, sb0, sb1, db0, db1, gb0, gb1,
            gs0, gs1, cs0, cs1, acc):
    cid = lax.axis_index("c")
    sid = lax.axis_index("s")
    wid = cid * NS + sid
    half = EDGES_PER_W // 2
    pltpu.sync_copy(src16_hbm.at[pl.ds(wid * half, half)], s16)
    pltpu.sync_copy(dst16_hbm.at[pl.ds(wid * half, half)], d16)
    _zero_fill(gb0, 16, HID)
    base = sid * ROWS_PER_TILE
    for t in range(ROWS_PER_TILE // 16):
        pltpu.sync_copy(gb0.at[pl.ds(0, 16)], acc.at[pl.ds(base + t * 16, 16)])
    plsc.subcore_barrier()

    def pair(jj, carry):
        j0 = 2 * jj
        _unpack_idx(s16, j0, sb0)
        g0 = pltpu.async_copy(xs_hbm.at[sb0], gb0, gs0)
        _unpack_idx(s16, j0 + 1, sb1)
        g1 = pltpu.async_copy(xs_hbm.at[sb1], gb1, gs1)
        _unpack_idx(d16, j0, db0)
        _unpack_idx(d16, j0 + 1, db1)
        g0.wait()
        s0 = pltpu.async_copy(gb0, acc.at[db0], cs0, add=True)
        g1.wait()
        s1 = pltpu.async_copy(gb1, acc.at[db1], cs1, add=True)
        s0.wait()
        s1.wait()
        return carry

    lax.fori_loop(0, CH_PER_W // 2, pair, 0)
    plsc.subcore_barrier()
    for t in range(ROWS_PER_TILE // CHUNK):
        pltpu.sync_copy(acc.at[pl.ds(base + t * CHUNK, CHUNK)], gb0)
        pltpu.sync_copy(gb0, out_hbm.at[cid, pl.ds(base + t * CHUNK, CHUNK)])


def _prep_body(degp_ref, w1p_ref, xs1_ref, nd_ref, ns_ref):
    d = degp_ref[0] + degp_ref[1]
    di = jnp.broadcast_to(d[:, 0:1], (NROW, W16))
    do = jnp.broadcast_to(d[:, 64:65], (NROW, W16))
    rows = lax.broadcasted_iota(jnp.int32, di.shape, 0)
    valid = rows < N_NODES
    nsv = jnp.where(valid & (do > 0.0), lax.rsqrt(do), 0.0)
    ndv = jnp.where(valid & (di > 0.0), lax.rsqrt(di), 0.0)
    lane = lax.broadcasted_iota(jnp.int32, di.shape, 1)
    one = jnp.float32(1.0)
    zero = jnp.float32(0.0)
    h = jnp.where(
        lane == 0, di,
        jnp.where(lane == 1, jnp.where(di > 3.0, one, zero),
                  jnp.where(lane == 2, 3.0 / di,
                            jnp.where(lane == 3, jnp.where(di > 4.0, one, zero),
                                      zero))))
    hs = jnp.where(valid, h * nsv, 0.0)
    xs1_ref[...] = jnp.dot(hs, w1p_ref[...], preferred_element_type=jnp.float32)
    nd_ref[...] = ndv
    ns_ref[...] = nsv


_prep_tc = pl.pallas_call(
    _prep_body,
    out_shape=(
        jax.ShapeDtypeStruct((NROW, HID), jnp.float32),
        jax.ShapeDtypeStruct((NROW, W16), jnp.float32),
        jax.ShapeDtypeStruct((NROW, W16), jnp.float32),
    ),
)


def _layer1_body(p_ref, nd_ref, ns_ref, b_ref, xs_ref):
    agg = (p_ref[0] + p_ref[1]) * nd_ref[:, 0:1]
    xs_ref[...] = jnp.tanh(agg + b_ref[...]) * ns_ref[:, 0:1]


_layer1_tc = pl.pallas_call(
    _layer1_body,
    out_shape=jax.ShapeDtypeStruct((NROW, HID), jnp.float32),
)


def _layer_body(p_ref, nd_ref, ns_ref, w_ref, b_ref, xs_ref):
    agg = (p_ref[0] + p_ref[1]) * nd_ref[:, 0:1]
    h = jnp.tanh(jnp.dot(agg, w_ref[...],
                         preferred_element_type=jnp.float32) + b_ref[...])
    xs_ref[...] = h * ns_ref[:, 0:1]


_layer_tc128 = pl.pallas_call(
    _layer_body,
    out_shape=jax.ShapeDtypeStruct((NROW, HID), jnp.float32),
)


def _final_body(p_ref, nd_ref, w_ref, b_ref, l1w_ref, l1b_ref, l2w_ref,
                l2b_ref, h_ref, g_ref, pred_ref):
    agg = (p_ref[0] + p_ref[1]) * nd_ref[:, 0:1]
    h = jnp.tanh(jnp.dot(agg, w_ref[...],
                         preferred_element_type=jnp.float32) + b_ref[...])
    h_ref[...] = h
    rows = lax.broadcasted_iota(jnp.int32, h.shape, 0)
    hm = jnp.where(rows < N_NODES, h, 0.0)
    g = jnp.sum(hm, axis=0, keepdims=True) * jnp.float32(1.0 / N_NODES)
    g_ref[...] = g
    e = jnp.dot(g, l1w_ref[...], preferred_element_type=jnp.float32) + l1b_ref[...]
    e = jnp.where(e > 0.0, e, 0.01 * e)
    z = jnp.sum(e * l2w_ref[...]) + l2b_ref[0, 0]
    pred_ref[...] = jnp.reshape(1.0 / (1.0 + jnp.exp(-z)), (1, 1))


_final_tc = pl.pallas_call(
    _final_body,
    out_shape=(
        jax.ShapeDtypeStruct((NROW, HID), jnp.float32),
        jax.ShapeDtypeStruct((1, HID), jnp.float32),
        jax.ShapeDtypeStruct((1, 1), jnp.float32),
    ),
)


def kernel(edge_index, w1, b1, w2, b2, w3, b3, w4, b4, w5, b5,
           l1_w, l1_b, l2_w, l2_b):
    src = edge_index[0]
    dst = edge_index[1]
    n_pad = E_PAD - N_EDGES
    pad_idx = (jnp.arange(n_pad, dtype=jnp.int32) % (NROW - N_NODES)) + N_NODES
    src_fl = jnp.concatenate([src, pad_idx])
    dst_fl = jnp.concatenate([dst, pad_idx])
    src16 = src_fl[0::2] | (src_fl[1::2] << 16)
    dst16 = dst_fl[0::2] | (dst_fl[1::2] << 16)

    w1p = jnp.zeros((W16, HID), jnp.float32).at[:4].set(w1)
    b1r = b1.reshape(1, HID)
    b2r = b2.reshape(1, HID)
    b3r = b3.reshape(1, HID)
    b4r = b4.reshape(1, HID)
    b5r = b5.reshape(1, HID)
    l1br = l1_b.reshape(1, HID2)
    l2wr = l2_w.reshape(1, HID2)
    l2br = l2_b.reshape(1, 1)

    deg_p = _deg_kernel(src16, dst16)
    xs1, nd, ns = _prep_tc(deg_p, w1p)

    agg1 = _agg128(src16, dst16, xs1)
    xs2 = _layer1_tc(agg1, nd, ns, b1r)
    agg2 = _agg128(src16, dst16, xs2)
    xs3 = _layer_tc128(agg2, nd, ns, w2, b2r)
    agg3 = _agg128(src16, dst16, xs3)
    xs4 = _layer_tc128(agg3, nd, ns, w3, b3r)
    agg4 = _agg128(src16, dst16, xs4)
    xs5 = _layer_tc128(agg4, nd, ns, w4, b4r)
    agg5 = _agg128(src16, dst16, xs5)
    h_full, graph_emb, pred = _final_tc(agg5, nd, w5, b5r, l1_w, l1br,
                                        l2wr, l2br)
    h_co = h_full[:N_NODES]
    return (pred, graph_emb, h_co)


# speedup vs baseline: 8.0411x; 1.0115x over previous
import functools

import jax
import jax.numpy as jnp
from jax import lax
from jax.experimental import pallas as pl
from jax.experimental.pallas import tpu as pltpu
from jax.experimental.pallas import tpu_sc as plsc

N_NODES = 10000
N_EDGES = 320000
HID = 128
HID2 = 64
W16 = 16

NC = 2
NS = 16
NW = NC * NS
CHUNK = 128
CH_PER_W = 80
EDGES_PER_W = CHUNK * CH_PER_W
E_PAD = NW * EDGES_PER_W
NROW = NS * 640
ROWS_PER_TILE = NROW // NS

_mesh = plsc.VectorSubcoreMesh(core_axis_name="c", subcore_axis_name="s")


def _zero_fill(ref, rows, width):
    z = jnp.zeros((16,), jnp.float32)
    for i in range(rows):
        for k in range(width // 16):
            ref[i, pl.ds(16 * k, 16)] = z


def _unpack_idx(slab, j, buf):
    for k in range(CHUNK // 32):
        w = slab[pl.ds(j * (CHUNK // 2) + 16 * k, 16)]
        buf[pl.ds(32 * k, 16)] = w & 0xFFFF
        buf[pl.ds(32 * k + 16, 16)] = lax.shift_right_logical(w, 16)


@functools.partial(
    pl.kernel,
    out_type=jax.ShapeDtypeStruct((NC, NROW, HID), jnp.float32),
    mesh=_mesh,
    scratch_types=[
        pltpu.VMEM((EDGES_PER_W // 2,), jnp.int32),
        pltpu.VMEM((EDGES_PER_W // 2,), jnp.int32),
        pltpu.VMEM((CHUNK,), jnp.int32),
        pltpu.VMEM((CHUNK,), jnp.int32),
        pltpu.VMEM((CHUNK,), jnp.int32),
        pltpu.VMEM((CHUNK,), jnp.int32),
        pltpu.VMEM((CHUNK, HID), jnp.float32),
        pltpu.VMEM((CHUNK, HID), jnp.float32),
        pltpu.VMEM((16, HID), jnp.float32),
        pltpu.SemaphoreType.DMA,
        pltpu.SemaphoreType.DMA,
        pltpu.SemaphoreType.DMA,
        pltpu.SemaphoreType.DMA,
        pltpu.VMEM_SHARED((NROW, HID), jnp.float32),
    ],
)
def _deg_kernel(src16_hbm, dst16_hbm, deg_hbm,
                s16, d16, sb0, sb1, db0, db1, ones_lo, ones_hi, zb,
                m0, m1, m2, m3, acc):
    cid = lax.axis_index("c")
    sid = lax.axis_index("s")
    wid = cid * NS + sid
    half = EDGES_PER_W // 2
    pltpu.sync_copy(src16_hbm.at[pl.ds(wid * half, half)], s16)
    pltpu.sync_copy(dst16_hbm.at[pl.ds(wid * half, half)], d16)
    onev = jnp.ones((16,), jnp.float32)
    zerov = jnp.zeros((16,), jnp.float32)
    for i in range(CHUNK):
        for k in range(HID // 16):
            ones_lo[i, pl.ds(16 * k, 16)] = onev if k < 4 else zerov
            ones_hi[i, pl.ds(16 * k, 16)] = zerov if k < 4 else onev
    _zero_fill(zb, 16, HID)
    base = sid * ROWS_PER_TILE
    for t in range(ROWS_PER_TILE // 16):
        pltpu.sync_copy(zb, acc.at[pl.ds(base + t * 16, 16)])
    plsc.subcore_barrier()

    def drain(sem):
        pltpu.make_async_copy(deg_hbm.at[cid, pl.ds(0, CHUNK)], ones_lo, sem).wait()

    _unpack_idx(d16, 0, db0)
    pltpu.async_copy(ones_lo, acc.at[db0], m0, add=True)
    _unpack_idx(s16, 0, sb0)
    pltpu.async_copy(ones_hi, acc.at[sb0], m1, add=True)
    _unpack_idx(d16, 1, db1)
    pltpu.async_copy(ones_lo, acc.at[db1], m2, add=True)
    _unpack_idx(s16, 1, sb1)
    pltpu.async_copy(ones_hi, acc.at[sb1], m3, add=True)

    def pair(jj, carry):
        j0 = 2 * jj
        drain(m0)
        drain(m1)
        _unpack_idx(d16, j0, db0)
        pltpu.async_copy(ones_lo, acc.at[db0], m0, add=True)
        _unpack_idx(s16, j0, sb0)
        pltpu.async_copy(ones_hi, acc.at[sb0], m1, add=True)
        drain(m2)
        drain(m3)
        _unpack_idx(d16, j0 + 1, db1)
        pltpu.async_copy(ones_lo, acc.at[db1], m2, add=True)
        _unpack_idx(s16, j0 + 1, sb1)
        pltpu.async_copy(ones_hi, acc.at[sb1], m3, add=True)
        return carry

    lax.fori_loop(1, CH_PER_W // 2, pair, 0)
    drain(m0)
    drain(m1)
    drain(m2)
    drain(m3)
    plsc.subcore_barrier()
    for t in range(ROWS_PER_TILE // CHUNK):
        pltpu.sync_copy(acc.at[pl.ds(base + t * CHUNK, CHUNK)], ones_lo)
        pltpu.sync_copy(ones_lo, deg_hbm.at[cid, pl.ds(base + t * CHUNK, CHUNK)])


@functools.partial(
    pl.kernel,
    out_type=jax.ShapeDtypeStruct((NC, NROW, HID), jnp.float32),
    mesh=_mesh,
    scratch_types=[
        pltpu.VMEM((EDGES_PER_W // 2,), jnp.int32),
        pltpu.VMEM((EDGES_PER_W // 2,), jnp.int32),
        pltpu.VMEM((CHUNK,), jnp.int32),
        pltpu.VMEM((CHUNK,), jnp.int32),
        pltpu.VMEM((CHUNK,), jnp.int32),
        pltpu.VMEM((CHUNK,), jnp.int32),
        pltpu.VMEM((CHUNK, HID), jnp.float32),
        pltpu.VMEM((CHUNK, HID), jnp.float32),
        pltpu.SemaphoreType.DMA,
        pltpu.SemaphoreType.DMA,
        pltpu.SemaphoreType.DMA,
        pltpu.SemaphoreType.DMA,
        pltpu.VMEM_SHARED((NROW, HID), jnp.float32),
    ],
)
def _agg128(src16_hbm, dst16_hbm, xs_hbm, out_hbm,
            s16, d16, sb0, sb1, db0, db1, gb0, gb1,
            gs0, gs1, cs0, cs1, acc):
    cid = lax.axis_index("c")
    sid = lax.axis_index("s")
    wid = cid * NS + sid
    half = EDGES_PER_W // 2
    pltpu.sync_copy(src16_hbm.at[pl.ds(wid * half, half)], s16)
    pltpu.sync_copy(dst16_hbm.at[pl.ds(wid * half, half)], d16)
    _zero_fill(gb0, 16, HID)
    base = sid * ROWS_PER_TILE
    for t in range(ROWS_PER_TILE // 16):
        pltpu.sync_copy(gb0.at[pl.ds(0, 16)], acc.at[pl.ds(base + t * 16, 16)])
    plsc.subcore_barrier()

    def drain(sem):
        pltpu.make_async_copy(xs_hbm.at[pl.ds(0, CHUNK)], gb0, sem).wait()

    _unpack_idx(s16, 0, sb0)
    g0 = pltpu.async_copy(xs_hbm.at[sb0], gb0, gs0)
    _unpack_idx(s16, 1, sb1)
    g1 = pltpu.async_copy(xs_hbm.at[sb1], gb1, gs1)
    _unpack_idx(d16, 0, db0)
    _unpack_idx(d16, 1, db1)
    g0.wait()
    pltpu.async_copy(gb0, acc.at[db0], cs0, add=True)
    g1.wait()
    pltpu.async_copy(gb1, acc.at[db1], cs1, add=True)

    def pair(jj, carry):
        j0 = 2 * jj
        drain(cs0)
        _unpack_idx(s16, j0, sb0)
        _unpack_idx(d16, j0, db0)
        g0 = pltpu.async_copy(xs_hbm.at[sb0], gb0, gs0)
        drain(cs1)
        _unpack_idx(s16, j0 + 1, sb1)
        _unpack_idx(d16, j0 + 1, db1)
        g1 = pltpu.async_copy(xs_hbm.at[sb1], gb1, gs1)
        g0.wait()
        pltpu.async_copy(gb0, acc.at[db0], cs0, add=True)
        g1.wait()
        pltpu.async_copy(gb1, acc.at[db1], cs1, add=True)
        return carry

    lax.fori_loop(1, CH_PER_W // 2, pair, 0)
    drain(cs0)
    drain(cs1)
    plsc.subcore_barrier()
    for t in range(ROWS_PER_TILE // CHUNK):
        pltpu.sync_copy(acc.at[pl.ds(base + t * CHUNK, CHUNK)], gb0)
        pltpu.sync_copy(gb0, out_hbm.at[cid, pl.ds(base + t * CHUNK, CHUNK)])


def _prep_body(degp_ref, w1p_ref, xs1_ref, nd_ref, ns_ref):
    d = degp_ref[0] + degp_ref[1]
    di = jnp.broadcast_to(d[:, 0:1], (NROW, W16))
    do = jnp.broadcast_to(d[:, 64:65], (NROW, W16))
    rows = lax.broadcasted_iota(jnp.int32, di.shape, 0)
    valid = rows < N_NODES
    nsv = jnp.where(valid & (do > 0.0), lax.rsqrt(do), 0.0)
    ndv = jnp.where(valid & (di > 0.0), lax.rsqrt(di), 0.0)
    lane = lax.broadcasted_iota(jnp.int32, di.shape, 1)
    one = jnp.float32(1.0)
    zero = jnp.float32(0.0)
    h = jnp.where(
        lane == 0, di,
        jnp.where(lane == 1, jnp.where(di > 3.0, one, zero),
                  jnp.where(lane == 2, 3.0 / di,
                            jnp.where(lane == 3, jnp.where(di > 4.0, one, zero),
                                      zero))))
    hs = jnp.where(valid, h * nsv, 0.0)
    xs1_ref[...] = jnp.dot(hs, w1p_ref[...], preferred_element_type=jnp.float32)
    nd_ref[...] = ndv
    ns_ref[...] = nsv


_prep_tc = pl.pallas_call(
    _prep_body,
    out_shape=(
        jax.ShapeDtypeStruct((NROW, HID), jnp.float32),
        jax.ShapeDtypeStruct((NROW, W16), jnp.float32),
        jax.ShapeDtypeStruct((NROW, W16), jnp.float32),
    ),
)


def _layer1_body(p_ref, nd_ref, ns_ref, b_ref, xs_ref):
    agg = (p_ref[0] + p_ref[1]) * nd_ref[:, 0:1]
    xs_ref[...] = jnp.tanh(agg + b_ref[...]) * ns_ref[:, 0:1]


_layer1_tc = pl.pallas_call(
    _layer1_body,
    out_shape=jax.ShapeDtypeStruct((NROW, HID), jnp.float32),
)


def _layer_body(p_ref, nd_ref, ns_ref, w_ref, b_ref, xs_ref):
    agg = (p_ref[0] + p_ref[1]) * nd_ref[:, 0:1]
    h = jnp.tanh(jnp.dot(agg, w_ref[...],
                         preferred_element_type=jnp.float32) + b_ref[...])
    xs_ref[...] = h * ns_ref[:, 0:1]


_layer_tc128 = pl.pallas_call(
    _layer_body,
    out_shape=jax.ShapeDtypeStruct((NROW, HID), jnp.float32),
)


def _final_body(p_ref, nd_ref, w_ref, b_ref, l1w_ref, l1b_ref, l2w_ref,
                l2b_ref, h_ref, g_ref, pred_ref):
    agg = (p_ref[0] + p_ref[1]) * nd_ref[:, 0:1]
    h = jnp.tanh(jnp.dot(agg, w_ref[...],
                         preferred_element_type=jnp.float32) + b_ref[...])
    h_ref[...] = h
    rows = lax.broadcasted_iota(jnp.int32, h.shape, 0)
    hm = jnp.where(rows < N_NODES, h, 0.0)
    g = jnp.sum(hm, axis=0, keepdims=True) * jnp.float32(1.0 / N_NODES)
    g_ref[...] = g
    e = jnp.dot(g, l1w_ref[...], preferred_element_type=jnp.float32) + l1b_ref[...]
    e = jnp.where(e > 0.0, e, 0.01 * e)
    z = jnp.sum(e * l2w_ref[...]) + l2b_ref[0, 0]
    pred_ref[...] = jnp.reshape(1.0 / (1.0 + jnp.exp(-z)), (1, 1))


_final_tc = pl.pallas_call(
    _final_body,
    out_shape=(
        jax.ShapeDtypeStruct((NROW, HID), jnp.float32),
        jax.ShapeDtypeStruct((1, HID), jnp.float32),
        jax.ShapeDtypeStruct((1, 1), jnp.float32),
    ),
)


def kernel(edge_index, w1, b1, w2, b2, w3, b3, w4, b4, w5, b5,
           l1_w, l1_b, l2_w, l2_b):
    src = edge_index[0]
    dst = edge_index[1]
    n_pad = E_PAD - N_EDGES
    pad_idx = (jnp.arange(n_pad, dtype=jnp.int32) % (NROW - N_NODES)) + N_NODES
    src_fl = jnp.concatenate([src, pad_idx])
    dst_fl = jnp.concatenate([dst, pad_idx])
    src16 = src_fl[0::2] | (src_fl[1::2] << 16)
    dst16 = dst_fl[0::2] | (dst_fl[1::2] << 16)

    w1p = jnp.zeros((W16, HID), jnp.float32).at[:4].set(w1)
    b1r = b1.reshape(1, HID)
    b2r = b2.reshape(1, HID)
    b3r = b3.reshape(1, HID)
    b4r = b4.reshape(1, HID)
    b5r = b5.reshape(1, HID)
    l1br = l1_b.reshape(1, HID2)
    l2wr = l2_w.reshape(1, HID2)
    l2br = l2_b.reshape(1, 1)

    deg_p = _deg_kernel(src16, dst16)
    xs1, nd, ns = _prep_tc(deg_p, w1p)

    agg1 = _agg128(src16, dst16, xs1)
    xs2 = _layer1_tc(agg1, nd, ns, b1r)
    agg2 = _agg128(src16, dst16, xs2)
    xs3 = _layer_tc128(agg2, nd, ns, w2, b2r)
    agg3 = _agg128(src16, dst16, xs3)
    xs4 = _layer_tc128(agg3, nd, ns, w3, b3r)
    agg4 = _agg128(src16, dst16, xs4)
    xs5 = _layer_tc128(agg4, nd, ns, w4, b4r)
    agg5 = _agg128(src16, dst16, xs5)
    h_full, graph_emb, pred = _final_tc(agg5, nd, w5, b5r, l1_w, l1br,
                                        l2wr, l2br)
    h_co = h_full[:N_NODES]
    return (pred, graph_emb, h_co)


# speedup vs baseline: 9.3977x; 1.1687x over previous
import functools

import jax
import jax.numpy as jnp
from jax import lax
from jax.experimental import pallas as pl
from jax.experimental.pallas import tpu as pltpu
from jax.experimental.pallas import tpu_sc as plsc

N_NODES = 10000
N_EDGES = 320000
HID = 128
HID2 = 64
W16 = 16

NC = 2
NS = 16
NW = NC * NS
CHUNK = 128
CH_PER_W = 80
EDGES_PER_W = CHUNK * CH_PER_W
E_PAD = NW * EDGES_PER_W
NROW = NS * 640
ROWS_PER_TILE = NROW // NS

_mesh = plsc.VectorSubcoreMesh(core_axis_name="c", subcore_axis_name="s")


def _zero_fill(ref, rows, width):
    z = jnp.zeros((16,), jnp.float32)
    for i in range(rows):
        for k in range(width // 16):
            ref[i, pl.ds(16 * k, 16)] = z


ACH = 64
NCH_A = EDGES_PER_W // ACH


def _unpack_idx(slab, j, buf):
    n = buf.shape[0]
    for k in range(n // 32):
        w = slab[pl.ds(j * (n // 2) + 16 * k, 16)]
        buf[pl.ds(32 * k, 16)] = w & 0xFFFF
        buf[pl.ds(32 * k + 16, 16)] = lax.shift_right_logical(w, 16)


@functools.partial(
    pl.kernel,
    out_type=jax.ShapeDtypeStruct((NC, NROW, HID), jnp.float32),
    mesh=_mesh,
    scratch_types=[
        pltpu.VMEM((EDGES_PER_W // 2,), jnp.int32),
        pltpu.VMEM((EDGES_PER_W // 2,), jnp.int32),
        pltpu.VMEM((CHUNK,), jnp.int32),
        pltpu.VMEM((CHUNK,), jnp.int32),
        pltpu.VMEM((CHUNK,), jnp.int32),
        pltpu.VMEM((CHUNK,), jnp.int32),
        pltpu.VMEM((CHUNK, HID), jnp.float32),
        pltpu.VMEM((CHUNK, HID), jnp.float32),
        pltpu.VMEM((16, HID), jnp.float32),
        pltpu.SemaphoreType.DMA,
        pltpu.SemaphoreType.DMA,
        pltpu.SemaphoreType.DMA,
        pltpu.SemaphoreType.DMA,
        pltpu.VMEM_SHARED((NROW, HID), jnp.float32),
    ],
)
def _deg_kernel(src16_hbm, dst16_hbm, deg_hbm,
                s16, d16, sb0, sb1, db0, db1, ones_lo, ones_hi, zb,
                m0, m1, m2, m3, acc):
    cid = lax.axis_index("c")
    sid = lax.axis_index("s")
    wid = cid * NS + sid
    half = EDGES_PER_W // 2
    pltpu.sync_copy(src16_hbm.at[pl.ds(wid * half, half)], s16)
    pltpu.sync_copy(dst16_hbm.at[pl.ds(wid * half, half)], d16)
    onev = jnp.ones((16,), jnp.float32)
    zerov = jnp.zeros((16,), jnp.float32)
    for i in range(CHUNK):
        for k in range(HID // 16):
            ones_lo[i, pl.ds(16 * k, 16)] = onev if k < 4 else zerov
            ones_hi[i, pl.ds(16 * k, 16)] = zerov if k < 4 else onev
    _zero_fill(zb, 16, HID)
    base = sid * ROWS_PER_TILE
    for t in range(ROWS_PER_TILE // 16):
        pltpu.sync_copy(zb, acc.at[pl.ds(base + t * 16, 16)])
    plsc.subcore_barrier()

    def drain(sem):
        pltpu.make_async_copy(deg_hbm.at[cid, pl.ds(0, CHUNK)], ones_lo, sem).wait()

    _unpack_idx(d16, 0, db0)
    pltpu.async_copy(ones_lo, acc.at[db0], m0, add=True)
    _unpack_idx(s16, 0, sb0)
    pltpu.async_copy(ones_hi, acc.at[sb0], m1, add=True)
    _unpack_idx(d16, 1, db1)
    pltpu.async_copy(ones_lo, acc.at[db1], m2, add=True)
    _unpack_idx(s16, 1, sb1)
    pltpu.async_copy(ones_hi, acc.at[sb1], m3, add=True)

    def pair(jj, carry):
        j0 = 2 * jj
        drain(m0)
        drain(m1)
        _unpack_idx(d16, j0, db0)
        pltpu.async_copy(ones_lo, acc.at[db0], m0, add=True)
        _unpack_idx(s16, j0, sb0)
        pltpu.async_copy(ones_hi, acc.at[sb0], m1, add=True)
        drain(m2)
        drain(m3)
        _unpack_idx(d16, j0 + 1, db1)
        pltpu.async_copy(ones_lo, acc.at[db1], m2, add=True)
        _unpack_idx(s16, j0 + 1, sb1)
        pltpu.async_copy(ones_hi, acc.at[sb1], m3, add=True)
        return carry

    lax.fori_loop(1, CH_PER_W // 2, pair, 0)
    drain(m0)
    drain(m1)
    drain(m2)
    drain(m3)
    plsc.subcore_barrier()
    for t in range(ROWS_PER_TILE // CHUNK):
        pltpu.sync_copy(acc.at[pl.ds(base + t * CHUNK, CHUNK)], ones_lo)
        pltpu.sync_copy(ones_lo, deg_hbm.at[cid, pl.ds(base + t * CHUNK, CHUNK)])


@functools.partial(
    pl.kernel,
    out_type=jax.ShapeDtypeStruct((NC, NROW, HID), jnp.float32),
    mesh=_mesh,
    scratch_types=[
        pltpu.VMEM((EDGES_PER_W // 2,), jnp.int32),
        pltpu.VMEM((EDGES_PER_W // 2,), jnp.int32),
    ] + [pltpu.VMEM((ACH,), jnp.int32) for _ in range(4)]
      + [pltpu.VMEM((ACH,), jnp.int32) for _ in range(4)]
      + [pltpu.VMEM((ACH, HID), jnp.float32) for _ in range(4)]
      + [pltpu.SemaphoreType.DMA for _ in range(8)]
      + [
        pltpu.VMEM_SHARED((NROW, HID), jnp.float32),
    ],
)
def _agg128(src16_hbm, dst16_hbm, xs_hbm, out_hbm,
            s16, d16, sa0, sa1, sa2, sa3, da0, da1, da2, da3,
            ga0, ga1, ga2, ga3, g_s0, g_s1, g_s2, g_s3,
            c_s0, c_s1, c_s2, c_s3, acc):
    cid = lax.axis_index("c")
    sid = lax.axis_index("s")
    wid = cid * NS + sid
    half = EDGES_PER_W // 2
    pltpu.sync_copy(src16_hbm.at[pl.ds(wid * half, half)], s16)
    pltpu.sync_copy(dst16_hbm.at[pl.ds(wid * half, half)], d16)
    sb = (sa0, sa1, sa2, sa3)
    db = (da0, da1, da2, da3)
    gb = (ga0, ga1, ga2, ga3)
    gs = (g_s0, g_s1, g_s2, g_s3)
    cs = (c_s0, c_s1, c_s2, c_s3)
    _zero_fill(ga0, 16, HID)
    base = sid * ROWS_PER_TILE
    for t in range(ROWS_PER_TILE // 16):
        pltpu.sync_copy(ga0.at[pl.ds(0, 16)], acc.at[pl.ds(base + t * 16, 16)])
    plsc.subcore_barrier()

    def drain(b):
        pltpu.make_async_copy(xs_hbm.at[pl.ds(0, ACH)], gb[b], cs[b]).wait()

    def fetch(j, b):
        _unpack_idx(s16, j, sb[b])
        _unpack_idx(d16, j, db[b])
        return pltpu.async_copy(xs_hbm.at[sb[b]], gb[b], gs[b])

    gds = [fetch(b, b) for b in range(4)]
    for b in range(4):
        gds[b].wait()
        pltpu.async_copy(gb[b], acc.at[db[b]], cs[b], add=True)

    def quad(qq, carry):
        j0 = 4 * qq
        g2 = []
        for b in range(4):
            drain(b)
            g2.append(fetch(j0 + b, b))
        for b in range(4):
            g2[b].wait()
            pltpu.async_copy(gb[b], acc.at[db[b]], cs[b], add=True)
        return carry

    lax.fori_loop(1, NCH_A // 4, quad, 0)
    for b in range(4):
        drain(b)
    plsc.subcore_barrier()
    for t in range(ROWS_PER_TILE // ACH):
        pltpu.sync_copy(acc.at[pl.ds(base + t * ACH, ACH)], ga0)
        pltpu.sync_copy(ga0, out_hbm.at[cid, pl.ds(base + t * ACH, ACH)])


def _prep_body(degp_ref, w1p_ref, xs1_ref, nd_ref, ns_ref):
    d = degp_ref[0] + degp_ref[1]
    di = jnp.broadcast_to(d[:, 0:1], (NROW, W16))
    do = jnp.broadcast_to(d[:, 64:65], (NROW, W16))
    rows = lax.broadcasted_iota(jnp.int32, di.shape, 0)
    valid = rows < N_NODES
    nsv = jnp.where(valid & (do > 0.0), lax.rsqrt(do), 0.0)
    ndv = jnp.where(valid & (di > 0.0), lax.rsqrt(di), 0.0)
    lane = lax.broadcasted_iota(jnp.int32, di.shape, 1)
    one = jnp.float32(1.0)
    zero = jnp.float32(0.0)
    h = jnp.where(
        lane == 0, di,
        jnp.where(lane == 1, jnp.where(di > 3.0, one, zero),
                  jnp.where(lane == 2, 3.0 / di,
                            jnp.where(lane == 3, jnp.where(di > 4.0, one, zero),
                                      zero))))
    hs = jnp.where(valid, h * nsv, 0.0)
    xs1_ref[...] = jnp.dot(hs, w1p_ref[...], preferred_element_type=jnp.float32)
    nd_ref[...] = ndv
    ns_ref[...] = nsv


_prep_tc = pl.pallas_call(
    _prep_body,
    out_shape=(
        jax.ShapeDtypeStruct((NROW, HID), jnp.float32),
        jax.ShapeDtypeStruct((NROW, W16), jnp.float32),
        jax.ShapeDtypeStruct((NROW, W16), jnp.float32),
    ),
)


def _layer1_body(p_ref, nd_ref, ns_ref, b_ref, xs_ref):
    agg = (p_ref[0] + p_ref[1]) * nd_ref[:, 0:1]
    xs_ref[...] = jnp.tanh(agg + b_ref[...]) * ns_ref[:, 0:1]


_layer1_tc = pl.pallas_call(
    _layer1_body,
    out_shape=jax.ShapeDtypeStruct((NROW, HID), jnp.float32),
)


def _layer_body(p_ref, nd_ref, ns_ref, w_ref, b_ref, xs_ref):
    agg = (p_ref[0] + p_ref[1]) * nd_ref[:, 0:1]
    h = jnp.tanh(jnp.dot(agg, w_ref[...],
                         preferred_element_type=jnp.float32) + b_ref[...])
    xs_ref[...] = h * ns_ref[:, 0:1]


_layer_tc128 = pl.pallas_call(
    _layer_body,
    out_shape=jax.ShapeDtypeStruct((NROW, HID), jnp.float32),
)


def _final_body(p_ref, nd_ref, w_ref, b_ref, l1w_ref, l1b_ref, l2w_ref,
                l2b_ref, h_ref, g_ref, pred_ref):
    agg = (p_ref[0] + p_ref[1]) * nd_ref[:, 0:1]
    h = jnp.tanh(jnp.dot(agg, w_ref[...],
                         preferred_element_type=jnp.float32) + b_ref[...])
    h_ref[...] = h
    rows = lax.broadcasted_iota(jnp.int32, h.shape, 0)
    hm = jnp.where(rows < N_NODES, h, 0.0)
    g = jnp.sum(hm, axis=0, keepdims=True) * jnp.float32(1.0 / N_NODES)
    g_ref[...] = g
    e = jnp.dot(g, l1w_ref[...], preferred_element_type=jnp.float32) + l1b_ref[...]
    e = jnp.where(e > 0.0, e, 0.01 * e)
    z = jnp.sum(e * l2w_ref[...]) + l2b_ref[0, 0]
    pred_ref[...] = jnp.reshape(1.0 / (1.0 + jnp.exp(-z)), (1, 1))


_final_tc = pl.pallas_call(
    _final_body,
    out_shape=(
        jax.ShapeDtypeStruct((NROW, HID), jnp.float32),
        jax.ShapeDtypeStruct((1, HID), jnp.float32),
        jax.ShapeDtypeStruct((1, 1), jnp.float32),
    ),
)


def kernel(edge_index, w1, b1, w2, b2, w3, b3, w4, b4, w5, b5,
           l1_w, l1_b, l2_w, l2_b):
    src = edge_index[0]
    dst = edge_index[1]
    n_pad = E_PAD - N_EDGES
    pad_idx = (jnp.arange(n_pad, dtype=jnp.int32) % (NROW - N_NODES)) + N_NODES
    src_fl = jnp.concatenate([src, pad_idx])
    dst_fl = jnp.concatenate([dst, pad_idx])
    src16 = src_fl[0::2] | (src_fl[1::2] << 16)
    dst16 = dst_fl[0::2] | (dst_fl[1::2] << 16)

    w1p = jnp.zeros((W16, HID), jnp.float32).at[:4].set(w1)
    b1r = b1.reshape(1, HID)
    b2r = b2.reshape(1, HID)
    b3r = b3.reshape(1, HID)
    b4r = b4.reshape(1, HID)
    b5r = b5.reshape(1, HID)
    l1br = l1_b.reshape(1, HID2)
    l2wr = l2_w.reshape(1, HID2)
    l2br = l2_b.reshape(1, 1)

    deg_p = _deg_kernel(src16, dst16)
    xs1, nd, ns = _prep_tc(deg_p, w1p)

    agg1 = _agg128(src16, dst16, xs1)
    xs2 = _layer1_tc(agg1, nd, ns, b1r)
    agg2 = _agg128(src16, dst16, xs2)
    xs3 = _layer_tc128(agg2, nd, ns, w2, b2r)
    agg3 = _agg128(src16, dst16, xs3)
    xs4 = _layer_tc128(agg3, nd, ns, w3, b3r)
    agg4 = _agg128(src16, dst16, xs4)
    xs5 = _layer_tc128(agg4, nd, ns, w4, b4r)
    agg5 = _agg128(src16, dst16, xs5)
    h_full, graph_emb, pred = _final_tc(agg5, nd, w5, b5r, l1_w, l1br,
                                        l2wr, l2br)
    h_co = h_full[:N_NODES]
    return (pred, graph_emb, h_co)


# speedup vs baseline: 9.4028x; 1.0005x over previous
import functools

import jax
import jax.numpy as jnp
from jax import lax
from jax.experimental import pallas as pl
from jax.experimental.pallas import tpu as pltpu
from jax.experimental.pallas import tpu_sc as plsc

N_NODES = 10000
N_EDGES = 320000
HID = 128
HID2 = 64
W16 = 16

NC = 2
NS = 16
NW = NC * NS
CHUNK = 128
CH_PER_W = 80
EDGES_PER_W = CHUNK * CH_PER_W
E_PAD = NW * EDGES_PER_W
NROW = NS * 640
ROWS_PER_TILE = NROW // NS

_mesh = plsc.VectorSubcoreMesh(core_axis_name="c", subcore_axis_name="s")


def _zero_fill(ref, rows, width):
    z = jnp.zeros((16,), jnp.float32)
    for i in range(rows):
        for k in range(width // 16):
            ref[i, pl.ds(16 * k, 16)] = z


ACH = 64
NCH_A = EDGES_PER_W // ACH


def _unpack_idx(slab, j, buf):
    n = buf.shape[0]
    for k in range(n // 32):
        w = slab[pl.ds(j * (n // 2) + 16 * k, 16)]
        buf[pl.ds(32 * k, 16)] = w & 0xFFFF
        buf[pl.ds(32 * k + 16, 16)] = lax.shift_right_logical(w, 16)


@functools.partial(
    pl.kernel,
    out_type=jax.ShapeDtypeStruct((NC, NROW, HID), jnp.float32),
    mesh=_mesh,
    scratch_types=[
        pltpu.VMEM((EDGES_PER_W // 2,), jnp.int32),
        pltpu.VMEM((EDGES_PER_W // 2,), jnp.int32),
    ] + [pltpu.VMEM((ACH,), jnp.int32) for _ in range(4)]
      + [pltpu.VMEM((ACH,), jnp.int32) for _ in range(4)]
      + [
        pltpu.VMEM((ACH, HID), jnp.float32),
        pltpu.VMEM((ACH, HID), jnp.float32),
        pltpu.VMEM((16, HID), jnp.float32),
        pltpu.SemaphoreType.DMA,
        pltpu.SemaphoreType.DMA,
        pltpu.SemaphoreType.DMA,
        pltpu.SemaphoreType.DMA,
        pltpu.VMEM_SHARED((NROW, HID), jnp.float32),
    ],
)
def _deg_kernel(src16_hbm, dst16_hbm, deg_hbm,
                s16, d16, sa0, sa1, sa2, sa3, da0, da1, da2, da3,
                ones_lo, ones_hi, zb, m0, m1, m2, m3, acc):
    cid = lax.axis_index("c")
    sid = lax.axis_index("s")
    wid = cid * NS + sid
    half = EDGES_PER_W // 2
    pltpu.sync_copy(src16_hbm.at[pl.ds(wid * half, half)], s16)
    pltpu.sync_copy(dst16_hbm.at[pl.ds(wid * half, half)], d16)
    onev = jnp.ones((16,), jnp.float32)
    zerov = jnp.zeros((16,), jnp.float32)
    for i in range(ACH):
        for k in range(HID // 16):
            ones_lo[i, pl.ds(16 * k, 16)] = onev if k < 4 else zerov
            ones_hi[i, pl.ds(16 * k, 16)] = zerov if k < 4 else onev
    _zero_fill(zb, 16, HID)
    base = sid * ROWS_PER_TILE
    for t in range(ROWS_PER_TILE // 16):
        pltpu.sync_copy(zb, acc.at[pl.ds(base + t * 16, 16)])
    plsc.subcore_barrier()

    sb = (sa0, sa1, sa2, sa3)
    db = (da0, da1, da2, da3)
    ms = (m0, m1, m2, m3)

    def drain(b):
        pltpu.make_async_copy(deg_hbm.at[cid, pl.ds(0, ACH)], ones_lo, ms[b]).wait()

    def fire(j, b):
        _unpack_idx(d16, j, db[b])
        pltpu.async_copy(ones_lo, acc.at[db[b]], ms[b], add=True)
        _unpack_idx(s16, j, sb[b])
        pltpu.async_copy(ones_hi, acc.at[sb[b]], ms[b], add=True)

    for b in range(4):
        fire(b, b)

    def quad(qq, carry):
        j0 = 4 * qq
        for b in range(4):
            drain(b)
            drain(b)
            fire(j0 + b, b)
        return carry

    lax.fori_loop(1, NCH_A // 4, quad, 0)
    for b in range(4):
        drain(b)
        drain(b)
    plsc.subcore_barrier()
    for t in range(ROWS_PER_TILE // ACH):
        pltpu.sync_copy(acc.at[pl.ds(base + t * ACH, ACH)], ones_lo)
        pltpu.sync_copy(ones_lo, deg_hbm.at[cid, pl.ds(base + t * ACH, ACH)])


@functools.partial(
    pl.kernel,
    out_type=jax.ShapeDtypeStruct((NC, NROW, HID), jnp.float32),
    mesh=_mesh,
    scratch_types=[
        pltpu.VMEM((EDGES_PER_W // 2,), jnp.int32),
        pltpu.VMEM((EDGES_PER_W // 2,), jnp.int32),
    ] + [pltpu.VMEM((ACH,), jnp.int32) for _ in range(4)]
      + [pltpu.VMEM((ACH,), jnp.int32) for _ in range(4)]
      + [pltpu.VMEM((ACH, HID), jnp.float32) for _ in range(4)]
      + [pltpu.SemaphoreType.DMA for _ in range(8)]
      + [
        pltpu.VMEM_SHARED((NROW, HID), jnp.float32),
    ],
)
def _agg128(src16_hbm, dst16_hbm, xs_hbm, out_hbm,
            s16, d16, sa0, sa1, sa2, sa3, da0, da1, da2, da3,
            ga0, ga1, ga2, ga3, g_s0, g_s1, g_s2, g_s3,
            c_s0, c_s1, c_s2, c_s3, acc):
    cid = lax.axis_index("c")
    sid = lax.axis_index("s")
    wid = cid * NS + sid
    half = EDGES_PER_W // 2
    pltpu.sync_copy(src16_hbm.at[pl.ds(wid * half, half)], s16)
    pltpu.sync_copy(dst16_hbm.at[pl.ds(wid * half, half)], d16)
    sb = (sa0, sa1, sa2, sa3)
    db = (da0, da1, da2, da3)
    gb = (ga0, ga1, ga2, ga3)
    gs = (g_s0, g_s1, g_s2, g_s3)
    cs = (c_s0, c_s1, c_s2, c_s3)
    _zero_fill(ga0, 16, HID)
    base = sid * ROWS_PER_TILE
    for t in range(ROWS_PER_TILE // 16):
        pltpu.sync_copy(ga0.at[pl.ds(0, 16)], acc.at[pl.ds(base + t * 16, 16)])
    plsc.subcore_barrier()

    def drain(b):
        pltpu.make_async_copy(xs_hbm.at[pl.ds(0, ACH)], gb[b], cs[b]).wait()

    def fetch(j, b):
        _unpack_idx(s16, j, sb[b])
        _unpack_idx(d16, j, db[b])
        return pltpu.async_copy(xs_hbm.at[sb[b]], gb[b], gs[b])

    gds = [fetch(b, b) for b in range(4)]
    for b in range(4):
        gds[b].wait()
        pltpu.async_copy(gb[b], acc.at[db[b]], cs[b], add=True)

    def quad(qq, carry):
        j0 = 4 * qq
        g2 = []
        for b in range(4):
            drain(b)
            g2.append(fetch(j0 + b, b))
        for b in range(4):
            g2[b].wait()
            pltpu.async_copy(gb[b], acc.at[db[b]], cs[b], add=True)
        return carry

    lax.fori_loop(1, NCH_A // 4, quad, 0)
    for b in range(4):
        drain(b)
    plsc.subcore_barrier()
    for t in range(ROWS_PER_TILE // ACH):
        pltpu.sync_copy(acc.at[pl.ds(base + t * ACH, ACH)], ga0)
        pltpu.sync_copy(ga0, out_hbm.at[cid, pl.ds(base + t * ACH, ACH)])


def _prep_body(degp_ref, w1p_ref, xs1_ref, nd_ref, ns_ref):
    d = degp_ref[0] + degp_ref[1]
    di = jnp.broadcast_to(d[:, 0:1], (NROW, W16))
    do = jnp.broadcast_to(d[:, 64:65], (NROW, W16))
    rows = lax.broadcasted_iota(jnp.int32, di.shape, 0)
    valid = rows < N_NODES
    nsv = jnp.where(valid & (do > 0.0), lax.rsqrt(do), 0.0)
    ndv = jnp.where(valid & (di > 0.0), lax.rsqrt(di), 0.0)
    lane = lax.broadcasted_iota(jnp.int32, di.shape, 1)
    one = jnp.float32(1.0)
    zero = jnp.float32(0.0)
    h = jnp.where(
        lane == 0, di,
        jnp.where(lane == 1, jnp.where(di > 3.0, one, zero),
                  jnp.where(lane == 2, 3.0 / di,
                            jnp.where(lane == 3, jnp.where(di > 4.0, one, zero),
                                      zero))))
    hs = jnp.where(valid, h * nsv, 0.0)
    xs1_ref[...] = jnp.dot(hs, w1p_ref[...], preferred_element_type=jnp.float32)
    nd_ref[...] = ndv
    ns_ref[...] = nsv


_prep_tc = pl.pallas_call(
    _prep_body,
    out_shape=(
        jax.ShapeDtypeStruct((NROW, HID), jnp.float32),
        jax.ShapeDtypeStruct((NROW, W16), jnp.float32),
        jax.ShapeDtypeStruct((NROW, W16), jnp.float32),
    ),
)


def _layer1_body(p_ref, nd_ref, ns_ref, b_ref, xs_ref):
    agg = (p_ref[0] + p_ref[1]) * nd_ref[:, 0:1]
    xs_ref[...] = jnp.tanh(agg + b_ref[...]) * ns_ref[:, 0:1]


_layer1_tc = pl.pallas_call(
    _layer1_body,
    out_shape=jax.ShapeDtypeStruct((NROW, HID), jnp.float32),
)


def _layer_body(p_ref, nd_ref, ns_ref, w_ref, b_ref, xs_ref):
    agg = (p_ref[0] + p_ref[1]) * nd_ref[:, 0:1]
    h = jnp.tanh(jnp.dot(agg, w_ref[...],
                         preferred_element_type=jnp.float32) + b_ref[...])
    xs_ref[...] = h * ns_ref[:, 0:1]


_layer_tc128 = pl.pallas_call(
    _layer_body,
    out_shape=jax.ShapeDtypeStruct((NROW, HID), jnp.float32),
)


def _final_body(p_ref, nd_ref, w_ref, b_ref, l1w_ref, l1b_ref, l2w_ref,
                l2b_ref, h_ref, g_ref, pred_ref):
    agg = (p_ref[0] + p_ref[1]) * nd_ref[:, 0:1]
    h = jnp.tanh(jnp.dot(agg, w_ref[...],
                         preferred_element_type=jnp.float32) + b_ref[...])
    h_ref[...] = h
    rows = lax.broadcasted_iota(jnp.int32, h.shape, 0)
    hm = jnp.where(rows < N_NODES, h, 0.0)
    g = jnp.sum(hm, axis=0, keepdims=True) * jnp.float32(1.0 / N_NODES)
    g_ref[...] = g
    e = jnp.dot(g, l1w_ref[...], preferred_element_type=jnp.float32) + l1b_ref[...]
    e = jnp.where(e > 0.0, e, 0.01 * e)
    z = jnp.sum(e * l2w_ref[...]) + l2b_ref[0, 0]
    pred_ref[...] = jnp.reshape(1.0 / (1.0 + jnp.exp(-z)), (1, 1))


_final_tc = pl.pallas_call(
    _final_body,
    out_shape=(
        jax.ShapeDtypeStruct((NROW, HID), jnp.float32),
        jax.ShapeDtypeStruct((1, HID), jnp.float32),
        jax.ShapeDtypeStruct((1, 1), jnp.float32),
    ),
)


def kernel(edge_index, w1, b1, w2, b2, w3, b3, w4, b4, w5, b5,
           l1_w, l1_b, l2_w, l2_b):
    src = edge_index[0]
    dst = edge_index[1]
    n_pad = E_PAD - N_EDGES
    pad_idx = (jnp.arange(n_pad, dtype=jnp.int32) % (NROW - N_NODES)) + N_NODES
    src_fl = jnp.concatenate([src, pad_idx])
    dst_fl = jnp.concatenate([dst, pad_idx])
    src16 = src_fl[0::2] | (src_fl[1::2] << 16)
    dst16 = dst_fl[0::2] | (dst_fl[1::2] << 16)

    w1p = jnp.zeros((W16, HID), jnp.float32).at[:4].set(w1)
    b1r = b1.reshape(1, HID)
    b2r = b2.reshape(1, HID)
    b3r = b3.reshape(1, HID)
    b4r = b4.reshape(1, HID)
    b5r = b5.reshape(1, HID)
    l1br = l1_b.reshape(1, HID2)
    l2wr = l2_w.reshape(1, HID2)
    l2br = l2_b.reshape(1, 1)

    deg_p = _deg_kernel(src16, dst16)
    xs1, nd, ns = _prep_tc(deg_p, w1p)

    agg1 = _agg128(src16, dst16, xs1)
    xs2 = _layer1_tc(agg1, nd, ns, b1r)
    agg2 = _agg128(src16, dst16, xs2)
    xs3 = _layer_tc128(agg2, nd, ns, w2, b2r)
    agg3 = _agg128(src16, dst16, xs3)
    xs4 = _layer_tc128(agg3, nd, ns, w3, b3r)
    agg4 = _agg128(src16, dst16, xs4)
    xs5 = _layer_tc128(agg4, nd, ns, w4, b4r)
    agg5 = _agg128(src16, dst16, xs5)
    h_full, graph_emb, pred = _final_tc(agg5, nd, w5, b5r, l1_w, l1br,
                                        l2wr, l2br)
    h_co = h_full[:N_NODES]
    return (pred, graph_emb, h_co)


# speedup vs baseline: 9.4692x; 1.0071x over previous
import functools

import jax
import jax.numpy as jnp
from jax import lax
from jax.experimental import pallas as pl
from jax.experimental.pallas import tpu as pltpu
from jax.experimental.pallas import tpu_sc as plsc

N_NODES = 10000
N_EDGES = 320000
HID = 128
HID2 = 64
W16 = 16

NC = 2
NS = 16
NW = NC * NS
CHUNK = 128
CH_PER_W = 80
EDGES_PER_W = CHUNK * CH_PER_W
E_PAD = NW * EDGES_PER_W
NROW = NS * 640
ROWS_PER_TILE = NROW // NS

_mesh = plsc.VectorSubcoreMesh(core_axis_name="c", subcore_axis_name="s")


def _zero_fill(ref, rows, width):
    z = jnp.zeros((16,), jnp.float32)
    for i in range(rows):
        for k in range(width // 16):
            ref[i, pl.ds(16 * k, 16)] = z


ACH = 64
NCH_A = EDGES_PER_W // ACH
GCH = 32
NSLOT = 8
NCH_G = EDGES_PER_W // GCH


def _unpack_idx(slab, j, buf):
    n = buf.shape[0]
    for k in range(n // 32):
        w = slab[pl.ds(j * (n // 2) + 16 * k, 16)]
        buf[pl.ds(32 * k, 16)] = w & 0xFFFF
        buf[pl.ds(32 * k + 16, 16)] = lax.shift_right_logical(w, 16)


@functools.partial(
    pl.kernel,
    out_type=jax.ShapeDtypeStruct((NC, NROW, HID), jnp.float32),
    mesh=_mesh,
    scratch_types=[
        pltpu.VMEM((EDGES_PER_W // 2,), jnp.int32),
        pltpu.VMEM((EDGES_PER_W // 2,), jnp.int32),
    ] + [pltpu.VMEM((ACH,), jnp.int32) for _ in range(4)]
      + [pltpu.VMEM((ACH,), jnp.int32) for _ in range(4)]
      + [
        pltpu.VMEM((ACH, HID), jnp.float32),
        pltpu.VMEM((ACH, HID), jnp.float32),
        pltpu.VMEM((16, HID), jnp.float32),
        pltpu.SemaphoreType.DMA,
        pltpu.SemaphoreType.DMA,
        pltpu.SemaphoreType.DMA,
        pltpu.SemaphoreType.DMA,
        pltpu.VMEM_SHARED((NROW, HID), jnp.float32),
    ],
)
def _deg_kernel(src16_hbm, dst16_hbm, deg_hbm,
                s16, d16, sa0, sa1, sa2, sa3, da0, da1, da2, da3,
                ones_lo, ones_hi, zb, m0, m1, m2, m3, acc):
    cid = lax.axis_index("c")
    sid = lax.axis_index("s")
    wid = cid * NS + sid
    half = EDGES_PER_W // 2
    pltpu.sync_copy(src16_hbm.at[pl.ds(wid * half, half)], s16)
    pltpu.sync_copy(dst16_hbm.at[pl.ds(wid * half, half)], d16)
    onev = jnp.ones((16,), jnp.float32)
    zerov = jnp.zeros((16,), jnp.float32)
    for i in range(ACH):
        for k in range(HID // 16):
            ones_lo[i, pl.ds(16 * k, 16)] = onev if k < 4 else zerov
            ones_hi[i, pl.ds(16 * k, 16)] = zerov if k < 4 else onev
    _zero_fill(zb, 16, HID)
    base = sid * ROWS_PER_TILE
    for t in range(ROWS_PER_TILE // 16):
        pltpu.sync_copy(zb, acc.at[pl.ds(base + t * 16, 16)])
    plsc.subcore_barrier()

    sb = (sa0, sa1, sa2, sa3)
    db = (da0, da1, da2, da3)
    ms = (m0, m1, m2, m3)

    def drain(b):
        pltpu.make_async_copy(deg_hbm.at[cid, pl.ds(0, ACH)], ones_lo, ms[b]).wait()

    def fire(j, b):
        _unpack_idx(d16, j, db[b])
        pltpu.async_copy(ones_lo, acc.at[db[b]], ms[b], add=True)
        _unpack_idx(s16, j, sb[b])
        pltpu.async_copy(ones_hi, acc.at[sb[b]], ms[b], add=True)

    for b in range(4):
        fire(b, b)

    def quad(qq, carry):
        j0 = 4 * qq
        for b in range(4):
            drain(b)
            drain(b)
            fire(j0 + b, b)
        return carry

    lax.fori_loop(1, NCH_A // 4, quad, 0)
    for b in range(4):
        drain(b)
        drain(b)
    plsc.subcore_barrier()
    for t in range(ROWS_PER_TILE // ACH):
        pltpu.sync_copy(acc.at[pl.ds(base + t * ACH, ACH)], ones_lo)
        pltpu.sync_copy(ones_lo, deg_hbm.at[cid, pl.ds(base + t * ACH, ACH)])


@functools.partial(
    pl.kernel,
    out_type=jax.ShapeDtypeStruct((NC, NROW, HID), jnp.float32),
    mesh=_mesh,
    scratch_types=[
        pltpu.VMEM((EDGES_PER_W // 2,), jnp.int32),
        pltpu.VMEM((EDGES_PER_W // 2,), jnp.int32),
    ] + [pltpu.VMEM((GCH,), jnp.int32) for _ in range(NSLOT)]
      + [pltpu.VMEM((GCH,), jnp.int32) for _ in range(NSLOT)]
      + [pltpu.VMEM((GCH, HID), jnp.float32) for _ in range(NSLOT)]
      + [pltpu.SemaphoreType.DMA for _ in range(2 * NSLOT)]
      + [
        pltpu.VMEM_SHARED((NROW, HID), jnp.float32),
    ],
)
def _agg128(src16_hbm, dst16_hbm, xs_hbm, out_hbm, s16, d16, *rest):
    cid = lax.axis_index("c")
    sid = lax.axis_index("s")
    wid = cid * NS + sid
    half = EDGES_PER_W // 2
    pltpu.sync_copy(src16_hbm.at[pl.ds(wid * half, half)], s16)
    pltpu.sync_copy(dst16_hbm.at[pl.ds(wid * half, half)], d16)
    sb = rest[0:NSLOT]
    db = rest[NSLOT:2 * NSLOT]
    gb = rest[2 * NSLOT:3 * NSLOT]
    gs = rest[3 * NSLOT:4 * NSLOT]
    cs = rest[4 * NSLOT:5 * NSLOT]
    acc = rest[5 * NSLOT]
    _zero_fill(gb[0], 16, HID)
    base = sid * ROWS_PER_TILE
    for t in range(ROWS_PER_TILE // 16):
        pltpu.sync_copy(gb[0].at[pl.ds(0, 16)], acc.at[pl.ds(base + t * 16, 16)])
    plsc.subcore_barrier()

    def drain(b):
        pltpu.make_async_copy(xs_hbm.at[pl.ds(0, GCH)], gb[b], cs[b]).wait()

    def fetch(j, b):
        _unpack_idx(s16, j, sb[b])
        _unpack_idx(d16, j, db[b])
        return pltpu.async_copy(xs_hbm.at[sb[b]], gb[b], gs[b])

    gds = [fetch(b, b) for b in range(NSLOT)]
    for b in range(NSLOT):
        gds[b].wait()
        pltpu.async_copy(gb[b], acc.at[db[b]], cs[b], add=True)

    def quad(qq, carry):
        j0 = NSLOT * qq
        g2 = []
        for b in range(NSLOT):
            drain(b)
            g2.append(fetch(j0 + b, b))
        for b in range(NSLOT):
            g2[b].wait()
            pltpu.async_copy(gb[b], acc.at[db[b]], cs[b], add=True)
        return carry

    lax.fori_loop(1, NCH_G // NSLOT, quad, 0)
    for b in range(NSLOT):
        drain(b)
    plsc.subcore_barrier()
    for t in range(ROWS_PER_TILE // GCH):
        pltpu.sync_copy(acc.at[pl.ds(base + t * GCH, GCH)], gb[0])
        pltpu.sync_copy(gb[0], out_hbm.at[cid, pl.ds(base + t * GCH, GCH)])


def _prep_body(degp_ref, w1p_ref, xs1_ref, nd_ref, ns_ref):
    d = degp_ref[0] + degp_ref[1]
    di = jnp.broadcast_to(d[:, 0:1], (NROW, W16))
    do = jnp.broadcast_to(d[:, 64:65], (NROW, W16))
    rows = lax.broadcasted_iota(jnp.int32, di.shape, 0)
    valid = rows < N_NODES
    nsv = jnp.where(valid & (do > 0.0), lax.rsqrt(do), 0.0)
    ndv = jnp.where(valid & (di > 0.0), lax.rsqrt(di), 0.0)
    lane = lax.broadcasted_iota(jnp.int32, di.shape, 1)
    one = jnp.float32(1.0)
    zero = jnp.float32(0.0)
    h = jnp.where(
        lane == 0, di,
        jnp.where(lane == 1, jnp.where(di > 3.0, one, zero),
                  jnp.where(lane == 2, 3.0 / di,
                            jnp.where(lane == 3, jnp.where(di > 4.0, one, zero),
                                      zero))))
    hs = jnp.where(valid, h * nsv, 0.0)
    xs1_ref[...] = jnp.dot(hs, w1p_ref[...], preferred_element_type=jnp.float32)
    nd_ref[...] = ndv
    ns_ref[...] = nsv


_prep_tc = pl.pallas_call(
    _prep_body,
    out_shape=(
        jax.ShapeDtypeStruct((NROW, HID), jnp.float32),
        jax.ShapeDtypeStruct((NROW, W16), jnp.float32),
        jax.ShapeDtypeStruct((NROW, W16), jnp.float32),
    ),
)


def _layer1_body(p_ref, nd_ref, ns_ref, b_ref, xs_ref):
    agg = (p_ref[0] + p_ref[1]) * nd_ref[:, 0:1]
    xs_ref[...] = jnp.tanh(agg + b_ref[...]) * ns_ref[:, 0:1]


_layer1_tc = pl.pallas_call(
    _layer1_body,
    out_shape=jax.ShapeDtypeStruct((NROW, HID), jnp.float32),
)


def _layer_body(p_ref, nd_ref, ns_ref, w_ref, b_ref, xs_ref):
    agg = (p_ref[0] + p_ref[1]) * nd_ref[:, 0:1]
    h = jnp.tanh(jnp.dot(agg, w_ref[...],
                         preferred_element_type=jnp.float32) + b_ref[...])
    xs_ref[...] = h * ns_ref[:, 0:1]


_layer_tc128 = pl.pallas_call(
    _layer_body,
    out_shape=jax.ShapeDtypeStruct((NROW, HID), jnp.float32),
)


def _final_body(p_ref, nd_ref, w_ref, b_ref, l1w_ref, l1b_ref, l2w_ref,
                l2b_ref, h_ref, g_ref, pred_ref):
    agg = (p_ref[0] + p_ref[1]) * nd_ref[:, 0:1]
    h = jnp.tanh(jnp.dot(agg, w_ref[...],
                         preferred_element_type=jnp.float32) + b_ref[...])
    h_ref[...] = h
    rows = lax.broadcasted_iota(jnp.int32, h.shape, 0)
    hm = jnp.where(rows < N_NODES, h, 0.0)
    g = jnp.sum(hm, axis=0, keepdims=True) * jnp.float32(1.0 / N_NODES)
    g_ref[...] = g
    e = jnp.dot(g, l1w_ref[...], preferred_element_type=jnp.float32) + l1b_ref[...]
    e = jnp.where(e > 0.0, e, 0.01 * e)
    z = jnp.sum(e * l2w_ref[...]) + l2b_ref[0, 0]
    pred_ref[...] = jnp.reshape(1.0 / (1.0 + jnp.exp(-z)), (1, 1))


_final_tc = pl.pallas_call(
    _final_body,
    out_shape=(
        jax.ShapeDtypeStruct((NROW, HID), jnp.float32),
        jax.ShapeDtypeStruct((1, HID), jnp.float32),
        jax.ShapeDtypeStruct((1, 1), jnp.float32),
    ),
)


def kernel(edge_index, w1, b1, w2, b2, w3, b3, w4, b4, w5, b5,
           l1_w, l1_b, l2_w, l2_b):
    src = edge_index[0]
    dst = edge_index[1]
    n_pad = E_PAD - N_EDGES
    pad_idx = (jnp.arange(n_pad, dtype=jnp.int32) % (NROW - N_NODES)) + N_NODES
    src_fl = jnp.concatenate([src, pad_idx])
    dst_fl = jnp.concatenate([dst, pad_idx])
    src16 = src_fl[0::2] | (src_fl[1::2] << 16)
    dst16 = dst_fl[0::2] | (dst_fl[1::2] << 16)

    w1p = jnp.zeros((W16, HID), jnp.float32).at[:4].set(w1)
    b1r = b1.reshape(1, HID)
    b2r = b2.reshape(1, HID)
    b3r = b3.reshape(1, HID)
    b4r = b4.reshape(1, HID)
    b5r = b5.reshape(1, HID)
    l1br = l1_b.reshape(1, HID2)
    l2wr = l2_w.reshape(1, HID2)
    l2br = l2_b.reshape(1, 1)

    deg_p = _deg_kernel(src16, dst16)
    xs1, nd, ns = _prep_tc(deg_p, w1p)

    agg1 = _agg128(src16, dst16, xs1)
    xs2 = _layer1_tc(agg1, nd, ns, b1r)
    agg2 = _agg128(src16, dst16, xs2)
    xs3 = _layer_tc128(agg2, nd, ns, w2, b2r)
    agg3 = _agg128(src16, dst16, xs3)
    xs4 = _layer_tc128(agg3, nd, ns, w3, b3r)
    agg4 = _agg128(src16, dst16, xs4)
    xs5 = _layer_tc128(agg4, nd, ns, w4, b4r)
    agg5 = _agg128(src16, dst16, xs5)
    h_full, graph_emb, pred = _final_tc(agg5, nd, w5, b5r, l1_w, l1br,
                                        l2wr, l2br)
    h_co = h_full[:N_NODES]
    return (pred, graph_emb, h_co)
